# Initial kernel scaffold; baseline (speedup 1.0000x reference)
import jax
import jax.numpy as jnp
from jax.experimental import pallas as pl


def kernel(x, edge_index, W_in, b_in, W1, b1, W2, b2, W3, b3):
    raise NotImplementedError("write your pallas kernel here")


# speedup vs baseline: 8.4569x; 8.4569x over previous
import jax
import jax.numpy as jnp
from jax import lax
from jax.experimental import pallas as pl
from jax.experimental.pallas import tpu as pltpu
from jax.experimental.pallas import tpu_sc as plsc

N = 10000
D = 128
E = 320000
NEG_SLOPE = 0.01

NC, NS = 2, 16
NW = NC * NS
CHUNK = 128
EPW = E // NW
CPW = -(-EPW // CHUNK)
EPW_PAD = CPW * CHUNK
E_PAD = EPW_PAD * NW
ACC_ROWS = 10240
ROWS_PT = ACC_ROWS // NS


def _sc_deg_body(dst_hbm, ones_hbm, zeros_hbm, out_hbm, dst_v, ones_v, acc):
    c = lax.axis_index("c")
    s = lax.axis_index("s")
    wid = c * NS + s
    pltpu.sync_copy(dst_hbm.at[wid], dst_v)
    pltpu.sync_copy(ones_hbm, ones_v)
    pltpu.sync_copy(zeros_hbm, acc.at[pl.ds(s * ROWS_PT, ROWS_PT)])
    plsc.subcore_barrier()

    def body(j, carry):
        pltpu.sync_copy(ones_v, acc.at[dst_v.at[j]], add=True)
        return carry

    lax.fori_loop(0, CPW, body, 0)
    plsc.subcore_barrier()
    pltpu.sync_copy(acc.at[pl.ds(s * ROWS_PT, ROWS_PT)],
                    out_hbm.at[c, pl.ds(s * ROWS_PT, ROWS_PT)])


def _sc_calls():
    mesh = plsc.VectorSubcoreMesh(core_axis_name="c", subcore_axis_name="s",
                                  num_cores=NC, num_subcores=NS)
    deg_call = pl.kernel(
        _sc_deg_body,
        out_type=jax.ShapeDtypeStruct((NC, ACC_ROWS, 16), jnp.float32),
        mesh=mesh,
        scratch_types=[
            pltpu.VMEM((CPW, CHUNK), jnp.int32),
            pltpu.VMEM((CHUNK, 16), jnp.float32),
            pltpu.VMEM_SHARED((ACC_ROWS, 16), jnp.float32),
        ],
    )
    agg_call = pl.kernel(
        _sc_agg_body,
        out_type=jax.ShapeDtypeStruct((NC, ACC_ROWS, D), jnp.float32),
        mesh=mesh,
        scratch_types=[
            pltpu.VMEM((CPW, CHUNK), jnp.int32),
            pltpu.VMEM((CPW, CHUNK), jnp.int32),
            pltpu.VMEM((CHUNK, D), jnp.float32),
            pltpu.VMEM_SHARED((ACC_ROWS, D), jnp.float32),
            pltpu.SemaphoreType.DMA,
        ],
    )
    return deg_call, agg_call


def _sc_agg_body(g_hbm, src_hbm, dst_hbm, zeros_hbm, out_hbm,
                 src_v, dst_v, rows, acc, sem):
    c = lax.axis_index("c")
    s = lax.axis_index("s")
    wid = c * NS + s
    pltpu.sync_copy(src_hbm.at[wid], src_v)
    pltpu.sync_copy(dst_hbm.at[wid], dst_v)
    pltpu.sync_copy(zeros_hbm, acc.at[pl.ds(s * ROWS_PT, ROWS_PT)])
    plsc.subcore_barrier()

    def body(j, carry):
        pltpu.async_copy(g_hbm.at[src_v.at[j]], rows, sem).wait()
        pltpu.sync_copy(rows, acc.at[dst_v.at[j]], add=True)
        return carry

    lax.fori_loop(0, CPW, body, 0)
    plsc.subcore_barrier()
    pltpu.sync_copy(acc.at[pl.ds(s * ROWS_PT, ROWS_PT)],
                    out_hbm.at[c, pl.ds(s * ROWS_PT, ROWS_PT)])


BN = 1000


def _dis_block(d0_ref, d1_ref):
    deg = 1.0 + d0_ref[:, 0:1] + d1_ref[:, 0:1]
    return lax.rsqrt(deg)


def _tc0_body(x_ref, win_ref, bin_ref, w1_ref, d0_ref, d1_ref, g_ref):
    h = jnp.dot(x_ref[:], win_ref[:], preferred_element_type=jnp.float32)
    h = h + bin_ref[:]
    y = jnp.dot(h, w1_ref[:], preferred_element_type=jnp.float32)
    g_ref[:] = y * _dis_block(d0_ref, d1_ref)


def _tc_mid_body(a0_ref, a1_ref, g_ref, d0_ref, d1_ref, b_ref, w_ref, o_ref):
    dis = _dis_block(d0_ref, d1_ref)
    pre = (a0_ref[:] + a1_ref[:] + g_ref[:]) * dis + b_ref[:]
    h = jnp.where(pre >= 0, pre, NEG_SLOPE * pre)
    o_ref[:] = jnp.dot(h, w_ref[:], preferred_element_type=jnp.float32) * dis


def _tc_fin_body(a0_ref, a1_ref, g_ref, d0_ref, d1_ref, b_ref, o_ref):
    dis = _dis_block(d0_ref, d1_ref)
    o_ref[:] = (a0_ref[:] + a1_ref[:] + g_ref[:]) * dis + b_ref[:]


_spec_nd = pl.BlockSpec((BN, D), lambda i: (i, 0))
_spec_w = pl.BlockSpec((D, D), lambda i: (0, 0))
_spec_b = pl.BlockSpec((1, D), lambda i: (0, 0))
_spec_deg = pl.BlockSpec((BN, 16), lambda i: (i, 0))
_out_nd = jax.ShapeDtypeStruct((N, D), jnp.float32)

_tc0 = pl.pallas_call(
    _tc0_body,
    grid=(N // BN,),
    in_specs=[_spec_nd, _spec_w, _spec_b, _spec_w, _spec_deg, _spec_deg],
    out_specs=_spec_nd,
    out_shape=_out_nd,
)

_tc_mid = pl.pallas_call(
    _tc_mid_body,
    grid=(N // BN,),
    in_specs=[_spec_nd, _spec_nd, _spec_nd, _spec_deg, _spec_deg,
              _spec_b, _spec_w],
    out_specs=_spec_nd,
    out_shape=_out_nd,
)

_tc_fin = pl.pallas_call(
    _tc_fin_body,
    grid=(N // BN,),
    in_specs=[_spec_nd, _spec_nd, _spec_nd, _spec_deg, _spec_deg, _spec_b],
    out_specs=_spec_nd,
    out_shape=_out_nd,
)


def kernel(x, edge_index, W_in, b_in, W1, b1, W2, b2, W3, b3):
    src = edge_index[0].astype(jnp.int32)
    dst = edge_index[1].astype(jnp.int32)
    pad = E_PAD - E
    src_r = jnp.concatenate([src, jnp.zeros((pad,), jnp.int32)])
    dst_r = jnp.concatenate([dst, jnp.full((pad,), N, jnp.int32)])
    src_r = src_r.reshape(NW, CPW, CHUNK)
    dst_r = dst_r.reshape(NW, CPW, CHUNK)

    onesND = jnp.ones((N, D), jnp.float32)
    zD = jnp.zeros((ROWS_PT, D), jnp.float32)

    _deg_call, _agg_call = _sc_calls()
    degs = _agg_call(onesND, src_r, dst_r, zD)
    d0, d1 = degs[0, :N, :16], degs[1, :N, :16]

    b_in2, b12, b22, b32 = (b.reshape(1, D) for b in (b_in, b1, b2, b3))

    g1 = _tc0(x, W_in, b_in2, W1, d0, d1)
    a = _agg_call(g1, src_r, dst_r, zD)
    g2 = _tc_mid(a[0, :N], a[1, :N], g1, d0, d1, b12, W2)
    a = _agg_call(g2, src_r, dst_r, zD)
    g3 = _tc_mid(a[0, :N], a[1, :N], g2, d0, d1, b22, W3)
    a = _agg_call(g3, src_r, dst_r, zD)
    return _tc_fin(a[0, :N], a[1, :N], g3, d0, d1, b32)
